# Initial kernel scaffold; baseline (speedup 1.0000x reference)
import functools

import jax
import jax.numpy as jnp
from jax import lax
from jax.experimental import pallas as pl
from jax.experimental.pallas import tpu as pltpu
from jax.experimental.pallas import tpu_sc as plsc

NC = 2
NS = 16
L = 16
CHUNK = 112


def _round_up(v, m):
    return (v + m - 1) // m * m


def _mesh():
    return plsc.VectorSubcoreMesh(core_axis_name="c", subcore_axis_name="s")


@functools.lru_cache(maxsize=None)
def _make_deg(nch, NH):
    per = NH // NS

    @functools.partial(
        pl.kernel,
        out_type=jax.ShapeDtypeStruct((NC, NH), jnp.float32),
        mesh=_mesh(),
        scratch_types=[
            pltpu.VMEM((nch, CHUNK), jnp.int32),
            pltpu.VMEM((CHUNK,), jnp.float32),
            pltpu.VMEM((per,), jnp.float32),
            pltpu.VMEM_SHARED((NH,), jnp.float32),
        ],
    )
    def deg_k(dst_hbm, degp_hbm, didx, ones, zbuf, hist):
        c = lax.axis_index("c")
        s = lax.axis_index("s")
        w = s * NC + c

        def fill_ones(k, carry):
            ones[pl.ds(k * L, L)] = jnp.ones((L,), jnp.float32)
            return carry

        lax.fori_loop(0, CHUNK // L, fill_ones, 0)

        def fill_z(k, carry):
            zbuf[pl.ds(k * L, L)] = jnp.zeros((L,), jnp.float32)
            return carry

        lax.fori_loop(0, per // L, fill_z, 0)
        pltpu.sync_copy(zbuf, hist.at[pl.ds(s * per, per)])
        plsc.subcore_barrier()

        pltpu.sync_copy(dst_hbm.at[w], didx)

        def chunk_body(j, carry):
            pltpu.sync_copy(ones, hist.at[didx.at[j]], add=True)
            return carry

        lax.fori_loop(0, nch, chunk_body, 0)
        plsc.subcore_barrier()
        pltpu.sync_copy(hist.at[pl.ds(s * per, per)],
                        degp_hbm.at[c, pl.ds(s * per, per)])

    return deg_k


@functools.lru_cache(maxsize=None)
def _make_agg(N, D, nch, NACC):
    half = N // NC
    dummy = half
    zpt = NACC // NS
    ZR = 64 if zpt % 64 == 0 else 8
    rpt = _round_up((half + NS - 1) // NS, 8)

    @functools.partial(
        pl.kernel,
        out_type=jax.ShapeDtypeStruct((N, D), jnp.float32),
        mesh=_mesh(),
        scratch_types=[
            pltpu.VMEM((nch, CHUNK), jnp.int32),
            pltpu.VMEM((nch, CHUNK), jnp.int32),
            pltpu.VMEM((CHUNK, D), jnp.float32),
            pltpu.VMEM((ZR, D), jnp.float32),
            pltpu.VMEM_SHARED((NACC, D), jnp.float32),
            pltpu.SemaphoreType.DMA,
        ],
    )
    def agg_k(src_hbm, dst_hbm, h2_hbm, out_hbm, sidx, dloc, rows, zbuf, acc,
              sem):
        c = lax.axis_index("c")
        s = lax.axis_index("s")

        def fill_z(r, carry):
            def fill_lane(k, carry2):
                zbuf[r, pl.ds(k * L, L)] = jnp.zeros((L,), jnp.float32)
                return carry2

            return lax.fori_loop(0, D // L, fill_lane, carry)

        lax.fori_loop(0, ZR, fill_z, 0)

        def zero_acc(i, carry):
            pltpu.sync_copy(zbuf, acc.at[pl.ds(s * zpt + i * ZR, ZR)])
            return carry

        lax.fori_loop(0, zpt // ZR, zero_acc, 0)
        plsc.subcore_barrier()

        pltpu.sync_copy(src_hbm.at[s], sidx)
        pltpu.sync_copy(dst_hbm.at[s], dloc)

        def localize(j, carry):
            def lane_grp(k, carry2):
                d = dloc[j, pl.ds(k * L, L)]
                dl = d - c * half
                ok = (dl >= 0) & (dl < half)
                dloc[j, pl.ds(k * L, L)] = jnp.where(ok, dl, dummy)
                return carry2

            return lax.fori_loop(0, CHUNK // L, lane_grp, carry)

        lax.fori_loop(0, nch, localize, 0)

        def chunk_body(j, carry):
            pltpu.async_copy(h2_hbm.at[sidx.at[j]], rows, sem).wait()
            pltpu.sync_copy(rows, acc.at[dloc.at[j]], add=True)
            return carry

        lax.fori_loop(0, nch, chunk_body, 0)
        plsc.subcore_barrier()

        start = jnp.minimum(s * rpt, half - rpt)
        pltpu.sync_copy(acc.at[pl.ds(start, rpt)],
                        out_hbm.at[pl.ds(c * half + start, rpt)])

    return agg_k


def _tc1_body(x_ref, w_ref, d_ref, h2_ref):
    h = jnp.dot(x_ref[...], w_ref[...], preferred_element_type=jnp.float32)
    dis = lax.rsqrt(d_ref[:, 0:1] + d_ref[:, 1:2] + 1.0)
    h2_ref[...] = h * dis


def _tc2_body(a_ref, h2_ref, d_ref, b1_ref, w_ref, lb_ref, o_ref):
    dis = lax.rsqrt(d_ref[:, 0:1] + d_ref[:, 1:2] + 1.0)
    t = jnp.tanh((a_ref[...] + h2_ref[...]) * dis + b1_ref[...])
    o_ref[...] = (jnp.dot(t, w_ref[...], preferred_element_type=jnp.float32)
                  + lb_ref[...])


@functools.lru_cache(maxsize=None)
def _make_tc1(N, D_in, D_hid, BR):
    grid = (N + BR - 1) // BR
    return pl.pallas_call(
        _tc1_body,
        grid=(grid,),
        in_specs=[
            pl.BlockSpec((BR, D_in), lambda i: (i, 0)),
            pl.BlockSpec((D_in, D_hid), lambda i: (0, 0)),
            pl.BlockSpec((BR, 2), lambda i: (i, 0)),
        ],
        out_specs=pl.BlockSpec((BR, D_hid), lambda i: (i, 0)),
        out_shape=jax.ShapeDtypeStruct((N, D_hid), jnp.float32),
    )


@functools.lru_cache(maxsize=None)
def _make_tc2(N, D_hid, D_out, BR):
    grid = (N + BR - 1) // BR
    return pl.pallas_call(
        _tc2_body,
        grid=(grid,),
        in_specs=[
            pl.BlockSpec((BR, D_hid), lambda i: (i, 0)),
            pl.BlockSpec((BR, D_hid), lambda i: (i, 0)),
            pl.BlockSpec((BR, 2), lambda i: (i, 0)),
            pl.BlockSpec((1, D_hid), lambda i: (0, 0)),
            pl.BlockSpec((D_hid, D_out), lambda i: (0, 0)),
            pl.BlockSpec((1, D_out), lambda i: (0, 0)),
        ],
        out_specs=pl.BlockSpec((BR, D_out), lambda i: (i, 0)),
        out_shape=jax.ShapeDtypeStruct((N, D_out), jnp.float32),
    )


def kernel(x, edge_index, W1, b1, lin_W, lin_b):
    N, D_in = x.shape
    D_hid = W1.shape[1]
    D_out = lin_W.shape[1]
    E = edge_index.shape[1]

    src = edge_index[0]
    dst = edge_index[1]

    EP = _round_up(E, NC * NS * CHUNK)
    pad = EP - E
    if pad:
        src = jnp.concatenate([src, jnp.zeros((pad,), src.dtype)])
        dst = jnp.concatenate([dst, jnp.full((pad,), N, dst.dtype)])

    nch_deg = EP // (NC * NS * CHUNK)
    nch_agg = EP // (NS * CHUNK)
    NH = _round_up(N + 1, NS * L)
    NACC = _round_up(N // NC + 1, NS * 8)

    dst_deg = dst.reshape(NC * NS, nch_deg, CHUNK)
    src_agg = src.reshape(NS, nch_agg, CHUNK)
    dst_agg = dst.reshape(NS, nch_agg, CHUNK)

    degp = _make_deg(nch_deg, NH)(dst_deg)
    degt = degp[:, :N].T

    BR = 1000 if N % 1000 == 0 else N
    h2 = _make_tc1(N, D_in, D_hid, BR)(x, W1, degt)
    acc = _make_agg(N, D_hid, nch_agg, NACC)(src_agg, dst_agg, h2)
    out = _make_tc2(N, D_hid, D_out, BR)(
        acc, h2, degt, b1.reshape(1, D_hid), lin_W, lin_b.reshape(1, D_out))
    return out


# speedup vs baseline: 8.3552x; 8.3552x over previous
import functools

import jax
import jax.numpy as jnp
from jax import lax
from jax.experimental import pallas as pl
from jax.experimental.pallas import tpu as pltpu
from jax.experimental.pallas import tpu_sc as plsc

NC = 2
NS = 16
L = 16
CHUNK = 64
STAGE = 16


def _round_up(v, m):
    return (v + m - 1) // m * m


def _mesh():
    return plsc.VectorSubcoreMesh(core_axis_name="c", subcore_axis_name="s")


@functools.lru_cache(maxsize=None)
def _make_deg(nch, NH):
    @functools.partial(
        pl.kernel,
        out_type=jax.ShapeDtypeStruct((NC * NS, NH), jnp.float32),
        mesh=_mesh(),
        compiler_params=pltpu.CompilerParams(needs_layout_passes=False),
        scratch_types=[
            pltpu.VMEM((nch, CHUNK), jnp.int32),
            pltpu.VMEM((NH,), jnp.float32),
        ],
    )
    def deg_k(dst_hbm, degp_hbm, didx, hist):
        c = lax.axis_index("c")
        s = lax.axis_index("s")
        w = s * NC + c

        def fill_z(k, carry):
            hist[pl.ds(k * L, L)] = jnp.zeros((L,), jnp.float32)
            return carry

        lax.fori_loop(0, NH // L, fill_z, 0)

        pltpu.sync_copy(dst_hbm.at[w], didx)
        ones = jnp.ones((L,), jnp.float32)

        def chunk_body(j, carry):
            def lane_grp(k, carry2):
                idx = didx[j, pl.ds(k * L, L)]
                plsc.addupdate_scatter(hist, [idx], ones)
                return carry2

            return lax.fori_loop(0, CHUNK // L, lane_grp, carry)

        lax.fori_loop(0, nch, chunk_body, 0)
        pltpu.sync_copy(hist, degp_hbm.at[w])

    return deg_k


@functools.lru_cache(maxsize=None)
def _make_agg(N, D, nch, NROWS):
    zpt = NROWS // NS

    @functools.partial(
        pl.kernel,
        out_type=jax.ShapeDtypeStruct((NC, NROWS, D), jnp.float32),
        mesh=_mesh(),
        scratch_types=[
            pltpu.VMEM((STAGE, CHUNK), jnp.int32),
            pltpu.VMEM((STAGE, CHUNK), jnp.int32),
            pltpu.VMEM((CHUNK, D), jnp.float32),
            pltpu.SemaphoreType.DMA,
        ],
    )
    def agg_k(src_hbm, dst_hbm, h2_hbm, acc_hbm, sidx, didx, rows, sem):
        c = lax.axis_index("c")
        s = lax.axis_index("s")
        w = s * NC + c

        def fill_z(r, carry):
            def fill_lane(k, carry2):
                rows[r, pl.ds(k * L, L)] = jnp.zeros((L,), jnp.float32)
                return carry2

            return lax.fori_loop(0, D // L, fill_lane, carry)

        lax.fori_loop(0, CHUNK, fill_z, 0)

        def zero_acc(i, carry):
            pltpu.sync_copy(
                rows, acc_hbm.at[c, pl.ds(s * zpt + i * CHUNK, CHUNK)])
            return carry

        lax.fori_loop(0, zpt // CHUNK, zero_acc, 0)
        plsc.subcore_barrier()

        def stage_body(t, carry):
            pltpu.sync_copy(src_hbm.at[w, pl.ds(t * STAGE, STAGE)], sidx)
            pltpu.sync_copy(dst_hbm.at[w, pl.ds(t * STAGE, STAGE)], didx)

            def chunk_body(j, carry2):
                pltpu.async_copy(h2_hbm.at[sidx.at[j]], rows, sem).wait()
                pltpu.sync_copy(rows, acc_hbm.at[c].at[didx.at[j]], add=True)
                return carry2

            return lax.fori_loop(0, STAGE, chunk_body, carry)

        lax.fori_loop(0, nch // STAGE, stage_body, 0)

    return agg_k


def _tc1_body(x_ref, w_ref, d_ref, h2_ref):
    h = jnp.dot(x_ref[...], w_ref[...], preferred_element_type=jnp.float32)
    deg = jnp.sum(d_ref[...], axis=1, keepdims=True) + 1.0
    h2_ref[...] = h * lax.rsqrt(deg)


def _tc2_body(a0_ref, a1_ref, h2_ref, d_ref, b1_ref, w_ref, lb_ref, o_ref):
    deg = jnp.sum(d_ref[...], axis=1, keepdims=True) + 1.0
    dis = lax.rsqrt(deg)
    t = jnp.tanh((a0_ref[0] + a1_ref[0] + h2_ref[...]) * dis + b1_ref[...])
    o_ref[...] = (jnp.dot(t, w_ref[...], preferred_element_type=jnp.float32)
                  + lb_ref[...])


@functools.lru_cache(maxsize=None)
def _make_tc1(N, D_in, D_hid, BR):
    grid = (N + BR - 1) // BR
    return pl.pallas_call(
        _tc1_body,
        grid=(grid,),
        in_specs=[
            pl.BlockSpec((BR, D_in), lambda i: (i, 0)),
            pl.BlockSpec((D_in, D_hid), lambda i: (0, 0)),
            pl.BlockSpec((BR, NC * NS), lambda i: (i, 0)),
        ],
        out_specs=pl.BlockSpec((BR, D_hid), lambda i: (i, 0)),
        out_shape=jax.ShapeDtypeStruct((N, D_hid), jnp.float32),
    )


@functools.lru_cache(maxsize=None)
def _make_tc2(N, NROWS, D_hid, D_out, BR):
    grid = (N + BR - 1) // BR
    return pl.pallas_call(
        _tc2_body,
        grid=(grid,),
        in_specs=[
            pl.BlockSpec((1, BR, D_hid), lambda i: (0, i, 0)),
            pl.BlockSpec((1, BR, D_hid), lambda i: (1, i, 0)),
            pl.BlockSpec((BR, D_hid), lambda i: (i, 0)),
            pl.BlockSpec((BR, NC * NS), lambda i: (i, 0)),
            pl.BlockSpec((1, D_hid), lambda i: (0, 0)),
            pl.BlockSpec((D_hid, D_out), lambda i: (0, 0)),
            pl.BlockSpec((1, D_out), lambda i: (0, 0)),
        ],
        out_specs=pl.BlockSpec((BR, D_out), lambda i: (i, 0)),
        out_shape=jax.ShapeDtypeStruct((N, D_out), jnp.float32),
    )


def kernel(x, edge_index, W1, b1, lin_W, lin_b):
    N, D_in = x.shape
    D_hid = W1.shape[1]
    D_out = lin_W.shape[1]
    E = edge_index.shape[1]

    src = edge_index[0]
    dst = edge_index[1]

    EP = _round_up(E, NC * NS * STAGE * CHUNK)
    pad = EP - E
    if pad:
        src = jnp.concatenate([src, jnp.zeros((pad,), src.dtype)])
        dst = jnp.concatenate([dst, jnp.full((pad,), N, dst.dtype)])

    nch = EP // (NC * NS * CHUNK)
    NH = _round_up(N + 1, 128)
    NROWS = _round_up(N + 1, NS * CHUNK)

    src3 = src.reshape(NC * NS, nch, CHUNK)
    dst3 = dst.reshape(NC * NS, nch, CHUNK)

    degp = _make_deg(nch, NH)(dst3)
    degt = degp[:, :N].T

    BR = 1000 if N % 1000 == 0 else N
    h2 = _make_tc1(N, D_in, D_hid, BR)(x, W1, degt)
    acc = _make_agg(N, D_hid, nch, NROWS)(src3, dst3, h2)
    out = _make_tc2(N, NROWS, D_hid, D_out, BR)(
        acc, acc, h2, degt, b1.reshape(1, D_hid), lin_W,
        lin_b.reshape(1, D_out))
    return out
